# Initial kernel scaffold; baseline (speedup 1.0000x reference)
import jax
import jax.numpy as jnp
from jax.experimental import pallas as pl


def kernel(x, x_0, edge_index, w_att_l, w_att_r):
    raise NotImplementedError("write your pallas kernel here")


# speedup vs baseline: 21.8144x; 21.8144x over previous
import functools

import jax
import jax.numpy as jnp
from jax import lax
from jax.experimental import pallas as pl
from jax.experimental.pallas import tpu as pltpu
from jax.experimental.pallas import tpu_sc as plsc

_EPS = 0.1
_SC_PARAMS = pltpu.CompilerParams(needs_layout_passes=False)
_NC = 2
_NS = 16
_NW = _NC * _NS
_CH = 80


def _tanh(z):
    a = jnp.abs(z)
    em = jnp.exp(-2.0 * a)
    return jnp.sign(z) * (1.0 - em) / (1.0 + em)


def _deg_kernel(n, w, n_win, e_per_w, n_chunks, chunks_per_tile):
    mesh = plsc.VectorSubcoreMesh(core_axis_name="c", subcore_axis_name="s")

    @functools.partial(
        pl.kernel,
        out_type=jax.ShapeDtypeStruct((_NC * n,), jnp.float32),
        mesh=mesh,
        scratch_types=[
            pltpu.VMEM((w,), jnp.int32),
            pltpu.VMEM((w,), jnp.float32),
            pltpu.VMEM((_CH,), jnp.float32),
            pltpu.VMEM_SHARED((n,), jnp.float32),
        ],
        compiler_params=_SC_PARAMS,
    )
    def deg_kernel(col_hbm, out_hbm, idx_v, ones_v, buf_v, deg_sh):
        cid = lax.axis_index("c")
        sid = lax.axis_index("s")
        wid = cid * _NS + sid

        @pl.loop(0, w, step=16)
        def _(i):
            ones_v[pl.ds(i, 16)] = jnp.ones((16,), jnp.float32)

        @pl.loop(0, _CH, step=16)
        def _(i):
            buf_v[pl.ds(i, 16)] = jnp.zeros((16,), jnp.float32)

        @pl.loop(0, chunks_per_tile)
        def _(j):
            c = sid * chunks_per_tile + j

            @pl.when(c < n_chunks)
            def _():
                pltpu.sync_copy(buf_v, deg_sh.at[pl.ds(c * _CH, _CH)])

        plsc.subcore_barrier()

        @pl.loop(0, n_win)
        def _(iw):
            base = wid * e_per_w + iw * w
            pltpu.sync_copy(col_hbm.at[pl.ds(base, w)], idx_v)
            pltpu.sync_copy(ones_v, deg_sh.at[idx_v], add=True)

        plsc.subcore_barrier()

        @pl.loop(0, chunks_per_tile)
        def _(j):
            c = sid * chunks_per_tile + j

            @pl.when(c < n_chunks)
            def _():
                pltpu.sync_copy(deg_sh.at[pl.ds(c * _CH, _CH)], buf_v)
                pltpu.sync_copy(buf_v, out_hbm.at[pl.ds(cid * n + c * _CH, _CH)])

    return deg_kernel


def _scal_body(x_ref, wl_ref, wr_ref, degp_ref, al_ref, ar_ref, ds_ref):
    x = x_ref[...]
    al_ref[...] = jnp.sum(x * wl_ref[...][None, :], axis=1)
    ar_ref[...] = jnp.sum(x * wr_ref[...][None, :], axis=1)
    deg = degp_ref[0, :] + degp_ref[1, :] + 1.0
    ds_ref[...] = lax.rsqrt(deg)


def _edge_kernel(n, d, w, n_win, e_per_w, n_chunks, chunks_per_tile):
    mesh = plsc.VectorSubcoreMesh(core_axis_name="c", subcore_axis_name="s")

    @functools.partial(
        pl.kernel,
        out_type=jax.ShapeDtypeStruct((_NC, n, d), jnp.float32),
        mesh=mesh,
        scratch_types=[
            pltpu.VMEM((w,), jnp.int32),
            pltpu.VMEM((w,), jnp.int32),
            pltpu.VMEM((n,), jnp.float32),
            pltpu.VMEM((n,), jnp.float32),
            pltpu.VMEM((n,), jnp.float32),
            pltpu.VMEM((w,), jnp.float32),
            pltpu.VMEM((w, d), jnp.float32),
            pltpu.VMEM_SHARED((n, d), jnp.float32),
            pltpu.SemaphoreType.DMA,
        ],
        compiler_params=_SC_PARAMS,
    )
    def edge_kernel(row_hbm, col_hbm, x_hbm, al_hbm, ar_hbm, ds_hbm, out_hbm,
                    idx_r, idx_c, tab_al, tab_ar, tab_ds, coef, rows,
                    acc, sem):
        cid = lax.axis_index("c")
        sid = lax.axis_index("s")
        wid = cid * _NS + sid

        pltpu.sync_copy(al_hbm, tab_al)
        pltpu.sync_copy(ar_hbm, tab_ar)
        pltpu.sync_copy(ds_hbm, tab_ds)

        @pl.loop(0, _CH)
        def _(r):
            for k in range(d // 16):
                rows[r, pl.ds(k * 16, 16)] = jnp.zeros((16,), jnp.float32)

        @pl.loop(0, chunks_per_tile)
        def _(j):
            c = sid * chunks_per_tile + j

            @pl.when(c < n_chunks)
            def _():
                pltpu.sync_copy(rows, acc.at[pl.ds(c * _CH, _CH)])

        plsc.subcore_barrier()

        @pl.loop(0, n_win)
        def _(iw):
            base = wid * e_per_w + iw * w
            pltpu.sync_copy(row_hbm.at[pl.ds(base, w)], idx_r)
            pltpu.sync_copy(col_hbm.at[pl.ds(base, w)], idx_c)
            pltpu.async_copy(x_hbm.at[idx_r], rows, sem).wait()

            @pl.loop(0, w, step=16)
            def _(g):
                rv = idx_r[pl.ds(g, 16)]
                cv = idx_c[pl.ds(g, 16)]
                al = plsc.load_gather(tab_al, [rv])
                ar = plsc.load_gather(tab_ar, [cv])
                dr = plsc.load_gather(tab_ds, [rv])
                dc = plsc.load_gather(tab_ds, [cv])
                coef[pl.ds(g, 16)] = _tanh(al + ar) * dr * dc

            @pl.loop(0, w)
            def _(ie):
                cb = plsc.load_gather(coef, [jnp.zeros((16,), jnp.int32) + ie])
                for k in range(d // 16):
                    rows[ie, pl.ds(k * 16, 16)] = rows[ie, pl.ds(k * 16, 16)] * cb

            pltpu.sync_copy(rows, acc.at[idx_c], add=True)

        plsc.subcore_barrier()

        @pl.loop(0, chunks_per_tile)
        def _(j):
            c = sid * chunks_per_tile + j

            @pl.when(c < n_chunks)
            def _():
                pltpu.sync_copy(acc.at[pl.ds(c * _CH, _CH)], rows)
                pltpu.sync_copy(rows, out_hbm.at[cid, pl.ds(c * _CH, _CH)])

    return edge_kernel


def _final_body(p_ref, x_ref, x0_ref, al_ref, ar_ref, ds_ref, o_ref):
    dis = ds_ref[...]
    c = jnp.tanh(al_ref[...] + ar_ref[...]) * dis * dis
    o_ref[...] = (p_ref[0] + p_ref[1] + x_ref[...] * c[:, None]
                  + _EPS * x0_ref[...])


@jax.jit
def kernel(x, x_0, edge_index, w_att_l, w_att_r):
    n, d = x.shape
    e = edge_index.shape[1]
    e_per_w = e // _NW
    w = 80
    n_win = e_per_w // w
    n_chunks = n // _CH
    chunks_per_tile = -(-n_chunks // _NS)

    row = edge_index[0]
    col = edge_index[1]

    deg_flat = _deg_kernel(n, w, n_win, e_per_w, n_chunks, chunks_per_tile)(col)
    deg_parts = deg_flat.reshape(_NC, n)

    al, ar, ds = pl.pallas_call(
        _scal_body,
        out_shape=[jax.ShapeDtypeStruct((n,), jnp.float32)] * 3,
    )(x, w_att_l, w_att_r, deg_parts)

    parts = _edge_kernel(n, d, w, n_win, e_per_w, n_chunks, chunks_per_tile)(
        row, col, x, al, ar, ds)

    out = pl.pallas_call(
        _final_body,
        out_shape=jax.ShapeDtypeStruct((n, d), jnp.float32),
    )(parts, x, x_0, al, ar, ds)
    return out


# speedup vs baseline: 31.3057x; 1.4351x over previous
import functools

import jax
import jax.numpy as jnp
from jax import lax
from jax.experimental import pallas as pl
from jax.experimental.pallas import tpu as pltpu
from jax.experimental.pallas import tpu_sc as plsc

_EPS = 0.1
_NC = 2
_NS = 16
_NW = _NC * _NS
_CH = 80
_SC_PARAMS = pltpu.CompilerParams(needs_layout_passes=False)


def _tanh(z):
    a = jnp.abs(z)
    em = jnp.exp(-2.0 * a)
    return jnp.sign(z) * (1.0 - em) / (1.0 + em)


def _deg_kernel(n, w, n_win, e_per_w, n_chunks, chunks_per_tile):
    mesh = plsc.VectorSubcoreMesh(core_axis_name="c", subcore_axis_name="s")

    @functools.partial(
        pl.kernel,
        out_type=jax.ShapeDtypeStruct((_NC * n,), jnp.float32),
        mesh=mesh,
        scratch_types=[
            pltpu.VMEM((w,), jnp.int32),
            pltpu.VMEM((w,), jnp.float32),
            pltpu.VMEM((_CH,), jnp.float32),
            pltpu.VMEM_SHARED((n,), jnp.float32),
        ],
        compiler_params=_SC_PARAMS,
    )
    def deg_kernel(col_hbm, out_hbm, idx_v, ones_v, buf_v, deg_sh):
        cid = lax.axis_index("c")
        sid = lax.axis_index("s")
        wid = cid * _NS + sid

        @pl.loop(0, w, step=16)
        def _(i):
            ones_v[pl.ds(i, 16)] = jnp.ones((16,), jnp.float32)

        @pl.loop(0, _CH, step=16)
        def _(i):
            buf_v[pl.ds(i, 16)] = jnp.zeros((16,), jnp.float32)

        @pl.loop(0, chunks_per_tile)
        def _(j):
            c = sid * chunks_per_tile + j

            @pl.when(c < n_chunks)
            def _():
                pltpu.sync_copy(buf_v, deg_sh.at[pl.ds(c * _CH, _CH)])

        plsc.subcore_barrier()

        @pl.loop(0, n_win)
        def _(iw):
            base = wid * e_per_w + iw * w
            pltpu.sync_copy(col_hbm.at[pl.ds(base, w)], idx_v)
            pltpu.sync_copy(ones_v, deg_sh.at[idx_v], add=True)

        plsc.subcore_barrier()

        @pl.loop(0, chunks_per_tile)
        def _(j):
            c = sid * chunks_per_tile + j

            @pl.when(c < n_chunks)
            def _():
                pltpu.sync_copy(deg_sh.at[pl.ds(c * _CH, _CH)], buf_v)
                pltpu.sync_copy(buf_v, out_hbm.at[pl.ds(cid * n + c * _CH, _CH)])

    return deg_kernel


def _scal_body(x_ref, wl_ref, wr_ref, degp_ref, al_ref, ar_ref, ds_ref, xp_ref):
    x = x_ref[...]
    al_ref[...] = jnp.sum(x * wl_ref[...][None, :], axis=1)
    ar_ref[...] = jnp.sum(x * wr_ref[...][None, :], axis=1)
    deg = degp_ref[0, :] + degp_ref[1, :] + 1.0
    dis = lax.rsqrt(deg)
    ds_ref[...] = dis
    xp_ref[...] = x * dis[:, None]


def _edge_kernel(n, d, w, n_win, e_per_w, n_chunks, chunks_per_tile):
    mesh = plsc.VectorSubcoreMesh(core_axis_name="c", subcore_axis_name="s")

    @functools.partial(
        pl.kernel,
        out_type=jax.ShapeDtypeStruct((_NC, n, d), jnp.float32),
        mesh=mesh,
        scratch_types=[
            pltpu.VMEM((n,), jnp.float32),
            pltpu.VMEM((n,), jnp.float32),
            pltpu.VMEM((w,), jnp.int32),
            pltpu.VMEM((w,), jnp.int32),
            pltpu.VMEM((w,), jnp.int32),
            pltpu.VMEM((w,), jnp.int32),
            pltpu.VMEM((w,), jnp.int32),
            pltpu.VMEM((w,), jnp.int32),
            pltpu.VMEM((w,), jnp.float32),
            pltpu.VMEM((w,), jnp.float32),
            pltpu.VMEM((w, d), jnp.float32),
            pltpu.VMEM((w, d), jnp.float32),
            pltpu.VMEM_SHARED((n, d), jnp.float32),
            pltpu.SemaphoreType.DMA,
            pltpu.SemaphoreType.DMA,
            pltpu.SemaphoreType.DMA,
            pltpu.SemaphoreType.DMA,
        ],
        compiler_params=_SC_PARAMS,
    )
    def edge_kernel(row_hbm, col_hbm, xp_hbm, al_hbm, ar_hbm, out_hbm,
                    tab_al, tab_ar, ir0, ir1, ic0, ic1, si0, si1, cf0, cf1,
                    rw0, rw1, acc, smi0, smi1, smg0, smg1):
        cid = lax.axis_index("c")
        sid = lax.axis_index("s")
        wid = cid * _NS + sid
        ebase = wid * e_per_w

        bufs = ((ir0, ic0, si0, cf0, rw0, smi0, smg0),
                (ir1, ic1, si1, cf1, rw1, smi1, smg1))

        def start_idx(iw, b):
            ir, ic, _, _, _, smi, _ = bufs[b]
            base = ebase + iw * w
            pltpu.async_copy(row_hbm.at[pl.ds(base, w)], ir, smi)
            pltpu.async_copy(col_hbm.at[pl.ds(base, w)], ic, smi)

        def wait_idx(iw, b):
            ir, ic, _, _, _, smi, _ = bufs[b]
            base = ebase + iw * w
            pltpu.make_async_copy(row_hbm.at[pl.ds(base, w)], ir, smi).wait()
            pltpu.make_async_copy(col_hbm.at[pl.ds(base, w)], ic, smi).wait()

        def start_gather(b):
            ir, _, _, _, rw, _, smg = bufs[b]
            pltpu.async_copy(xp_hbm.at[ir], rw, smg)

        def wait_gather(b):
            ir, _, _, _, rw, _, smg = bufs[b]
            pltpu.make_async_copy(xp_hbm.at[ir], rw, smg).wait()

        pltpu.sync_copy(al_hbm, tab_al)
        pltpu.sync_copy(ar_hbm, tab_ar)

        @pl.loop(0, _CH)
        def _(r):
            for k in range(d // 16):
                rw0[r, pl.ds(k * 16, 16)] = jnp.zeros((16,), jnp.float32)

        @pl.loop(0, chunks_per_tile)
        def _(j):
            c = sid * chunks_per_tile + j

            @pl.when(c < n_chunks)
            def _():
                pltpu.sync_copy(rw0, acc.at[pl.ds(c * _CH, _CH)])

        plsc.subcore_barrier()

        start_idx(0, 0)
        start_idx(1, 1)
        wait_idx(0, 0)
        start_gather(0)

        def body(iw, b):
            ir, ic, si, cf, rw, _, _ = bufs[b]

            @pl.loop(0, w, step=16)
            def _(g):
                rv = ir[pl.ds(g, 16)]
                cv = ic[pl.ds(g, 16)]
                al = plsc.load_gather(tab_al, [rv])
                ar = plsc.load_gather(tab_ar, [cv])
                cf[pl.ds(g, 16)] = _tanh(al + ar)
                si[pl.ds(g, 16)] = cv

            wait_gather(b)

            @pl.when(iw + 2 < n_win)
            def _():
                start_idx(iw + 2, b)

            @pl.loop(0, w)
            def _(ie):
                cb = plsc.load_gather(cf, [jnp.zeros((16,), jnp.int32) + ie])
                for k in range(d // 16):
                    rw[ie, pl.ds(k * 16, 16)] = rw[ie, pl.ds(k * 16, 16)] * cb

            @pl.when(iw + 1 < n_win)
            def _():
                wait_idx(iw + 1, 1 - b)
                start_gather(1 - b)

            pltpu.sync_copy(rw, acc.at[si], add=True)

        @pl.loop(0, (n_win + 1) // 2)
        def _(i):
            for db in range(2):
                iw = 2 * i + db

                @pl.when(iw < n_win)
                def _():
                    body(iw, db)

        plsc.subcore_barrier()

        @pl.loop(0, chunks_per_tile)
        def _(j):
            c = sid * chunks_per_tile + j

            @pl.when(c < n_chunks)
            def _():
                pltpu.sync_copy(acc.at[pl.ds(c * _CH, _CH)], rw0)
                pltpu.sync_copy(rw0, out_hbm.at[cid, pl.ds(c * _CH, _CH)])

    return edge_kernel


def _final_body(p_ref, x_ref, x0_ref, al_ref, ar_ref, ds_ref, o_ref):
    dis = ds_ref[...]
    c = jnp.tanh(al_ref[...] + ar_ref[...]) * dis * dis
    o_ref[...] = ((p_ref[0] + p_ref[1]) * dis[:, None] + x_ref[...] * c[:, None]
                  + _EPS * x0_ref[...])


@jax.jit
def kernel(x, x_0, edge_index, w_att_l, w_att_r):
    n, d = x.shape
    e = edge_index.shape[1]
    e_per_w = e // _NW
    w = 80
    n_win = e_per_w // w
    n_chunks = n // _CH
    chunks_per_tile = -(-n_chunks // _NS)

    row = edge_index[0]
    col = edge_index[1]

    deg_flat = _deg_kernel(n, w, n_win, e_per_w, n_chunks, chunks_per_tile)(col)
    deg_parts = deg_flat.reshape(_NC, n)

    al, ar, ds, xp = pl.pallas_call(
        _scal_body,
        out_shape=[jax.ShapeDtypeStruct((n,), jnp.float32)] * 3
        + [jax.ShapeDtypeStruct((n, d), jnp.float32)],
    )(x, w_att_l, w_att_r, deg_parts)

    parts = _edge_kernel(n, d, w, n_win, e_per_w, n_chunks, chunks_per_tile)(
        row, col, xp, al, ar)

    out = pl.pallas_call(
        _final_body,
        out_shape=jax.ShapeDtypeStruct((n, d), jnp.float32),
    )(parts, x, x_0, al, ar, ds)
    return out


# speedup vs baseline: 35.7882x; 1.1432x over previous
import functools

import jax
import jax.numpy as jnp
from jax import lax
from jax.experimental import pallas as pl
from jax.experimental.pallas import tpu as pltpu
from jax.experimental.pallas import tpu_sc as plsc

_EPS = 0.1
_NC = 2
_NS = 16
_NW = _NC * _NS
_CH = 80
_SC_PARAMS = pltpu.CompilerParams(needs_layout_passes=False)


def _tanh(z):
    a = jnp.abs(z)
    em = jnp.exp(-2.0 * a)
    return jnp.sign(z) * (1.0 - em) / (1.0 + em)


def _deg_kernel(n, w, n_win, e_per_w):
    mesh = plsc.VectorSubcoreMesh(core_axis_name="c", subcore_axis_name="s")

    @functools.partial(
        pl.kernel,
        out_type=jax.ShapeDtypeStruct((_NW * n,), jnp.float32),
        mesh=mesh,
        scratch_types=[
            pltpu.VMEM((w,), jnp.int32),
            pltpu.VMEM((w,), jnp.int32),
            pltpu.VMEM((n,), jnp.float32),
            pltpu.SemaphoreType.DMA,
            pltpu.SemaphoreType.DMA,
        ],
        compiler_params=_SC_PARAMS,
    )
    def deg_kernel(col_hbm, out_hbm, idx0, idx1, hist, smi0, smi1):
        cid = lax.axis_index("c")
        sid = lax.axis_index("s")
        wid = cid * _NS + sid
        ebase = wid * e_per_w
        bufs = ((idx0, smi0), (idx1, smi1))
        ones16 = jnp.ones((16,), jnp.float32)

        @pl.loop(0, n, step=128)
        def _(i):
            for k in range(8):
                hist[pl.ds(i + k * 16, 16)] = jnp.zeros((16,), jnp.float32)

        idx, smi = bufs[0]
        pltpu.async_copy(col_hbm.at[pl.ds(ebase, w)], idx, smi)

        def body(iw, b):
            idx, smi = bufs[b]
            base = ebase + iw * w
            pltpu.make_async_copy(col_hbm.at[pl.ds(base, w)], idx, smi).wait()

            @pl.when(iw + 1 < n_win)
            def _():
                nidx, nsmi = bufs[1 - b]
                nbase = ebase + (iw + 1) * w
                pltpu.async_copy(col_hbm.at[pl.ds(nbase, w)], nidx, nsmi)

            @pl.loop(0, w, step=16)
            def _(g):
                plsc.addupdate_scatter(hist, [idx[pl.ds(g, 16)]], ones16)

        @pl.loop(0, (n_win + 1) // 2)
        def _(i):
            for db in range(2):
                iw = 2 * i + db

                @pl.when(iw < n_win)
                def _():
                    body(iw, db)

        pltpu.sync_copy(hist, out_hbm.at[pl.ds(wid * n, n)])

    return deg_kernel


def _scal_body(x_ref, wl_ref, wr_ref, degp_ref, al_ref, ar_ref, ds_ref, xp_ref):
    x = x_ref[...]
    al_ref[...] = jnp.sum(x * wl_ref[...][None, :], axis=1)
    ar_ref[...] = jnp.sum(x * wr_ref[...][None, :], axis=1)
    deg = jnp.sum(degp_ref[...], axis=0) + 1.0
    dis = lax.rsqrt(deg)
    ds_ref[...] = dis
    xp_ref[...] = x * dis[:, None]


def _edge_kernel(n, d, w, n_win, e_per_w, n_chunks, chunks_per_tile):
    mesh = plsc.VectorSubcoreMesh(core_axis_name="c", subcore_axis_name="s")

    @functools.partial(
        pl.kernel,
        out_type=jax.ShapeDtypeStruct((_NC, n, d), jnp.float32),
        mesh=mesh,
        scratch_types=[
            pltpu.VMEM((n,), jnp.float32),
            pltpu.VMEM((n,), jnp.float32),
            pltpu.VMEM((w,), jnp.int32),
            pltpu.VMEM((w,), jnp.int32),
            pltpu.VMEM((w,), jnp.int32),
            pltpu.VMEM((w,), jnp.int32),
            pltpu.VMEM((w, d), jnp.float32),
            pltpu.VMEM((w, d), jnp.float32),
            pltpu.VMEM_SHARED((n, d), jnp.float32),
            pltpu.SemaphoreType.DMA,
            pltpu.SemaphoreType.DMA,
            pltpu.SemaphoreType.DMA,
            pltpu.SemaphoreType.DMA,
        ],
        compiler_params=_SC_PARAMS,
    )
    def edge_kernel(row_hbm, col_hbm, xp_hbm, al_hbm, ar_hbm, out_hbm,
                    tab_al, tab_ar, ir0, ir1, ic0, ic1,
                    rw0, rw1, acc, smi0, smi1, smg0, smg1):
        cid = lax.axis_index("c")
        sid = lax.axis_index("s")
        wid = cid * _NS + sid
        ebase = wid * e_per_w

        bufs = ((ir0, ic0, rw0, smi0, smg0),
                (ir1, ic1, rw1, smi1, smg1))

        def start_idx(iw, b):
            ir, ic, _, smi, _ = bufs[b]
            base = ebase + iw * w
            pltpu.async_copy(row_hbm.at[pl.ds(base, w)], ir, smi)
            pltpu.async_copy(col_hbm.at[pl.ds(base, w)], ic, smi)

        def wait_idx(iw, b):
            ir, ic, _, smi, _ = bufs[b]
            base = ebase + iw * w
            pltpu.make_async_copy(row_hbm.at[pl.ds(base, w)], ir, smi).wait()
            pltpu.make_async_copy(col_hbm.at[pl.ds(base, w)], ic, smi).wait()

        def start_gather(b):
            ir, _, rw, _, smg = bufs[b]
            pltpu.async_copy(xp_hbm.at[ir], rw, smg)

        def wait_gather(b):
            ir, _, rw, _, smg = bufs[b]
            pltpu.make_async_copy(xp_hbm.at[ir], rw, smg).wait()

        pltpu.sync_copy(al_hbm, tab_al)
        pltpu.sync_copy(ar_hbm, tab_ar)

        @pl.loop(0, _CH)
        def _(r):
            for k in range(d // 16):
                rw0[r, pl.ds(k * 16, 16)] = jnp.zeros((16,), jnp.float32)

        @pl.loop(0, chunks_per_tile)
        def _(j):
            c = sid * chunks_per_tile + j

            @pl.when(c < n_chunks)
            def _():
                pltpu.sync_copy(rw0, acc.at[pl.ds(c * _CH, _CH)])

        plsc.subcore_barrier()

        start_idx(0, 0)
        start_idx(1, 1)
        wait_idx(0, 0)
        start_gather(0)

        def body(iw, b):
            ir, ic, rw, _, _ = bufs[b]

            wait_gather(b)

            @pl.loop(0, w, step=16)
            def _(g):
                rv = ir[pl.ds(g, 16)]
                cv = ic[pl.ds(g, 16)]
                al = plsc.load_gather(tab_al, [rv])
                ar = plsc.load_gather(tab_ar, [cv])
                t = _tanh(al + ar)
                for j in range(16):
                    cb = t.at[jnp.full((16,), j, jnp.int32)].get(
                        mode="promise_in_bounds")
                    for k in range(d // 16):
                        rw[g + j, pl.ds(k * 16, 16)] = (
                            rw[g + j, pl.ds(k * 16, 16)] * cb)

            @pl.when(iw + 1 < n_win)
            def _():
                wait_idx(iw + 1, 1 - b)
                start_gather(1 - b)

            pltpu.sync_copy(rw, acc.at[ic], add=True)

            @pl.when(iw + 2 < n_win)
            def _():
                start_idx(iw + 2, b)

        @pl.loop(0, (n_win + 1) // 2)
        def _(i):
            for db in range(2):
                iw = 2 * i + db

                @pl.when(iw < n_win)
                def _():
                    body(iw, db)

        plsc.subcore_barrier()

        @pl.loop(0, chunks_per_tile)
        def _(j):
            c = sid * chunks_per_tile + j

            @pl.when(c < n_chunks)
            def _():
                pltpu.sync_copy(acc.at[pl.ds(c * _CH, _CH)], rw0)
                pltpu.sync_copy(rw0, out_hbm.at[cid, pl.ds(c * _CH, _CH)])

    return edge_kernel


def _final_body(p_ref, x_ref, x0_ref, al_ref, ar_ref, ds_ref, o_ref):
    dis = ds_ref[...]
    c = jnp.tanh(al_ref[...] + ar_ref[...]) * dis * dis
    o_ref[...] = ((p_ref[0] + p_ref[1]) * dis[:, None] + x_ref[...] * c[:, None]
                  + _EPS * x0_ref[...])


@jax.jit
def kernel(x, x_0, edge_index, w_att_l, w_att_r):
    n, d = x.shape
    e = edge_index.shape[1]
    e_per_w = e // _NW
    w = 80
    n_win = e_per_w // w
    n_chunks = n // _CH
    chunks_per_tile = -(-n_chunks // _NS)

    row = edge_index[0]
    col = edge_index[1]

    deg_flat = _deg_kernel(n, w, n_win, e_per_w)(col)
    deg_parts = deg_flat.reshape(_NW, n)

    al, ar, ds, xp = pl.pallas_call(
        _scal_body,
        out_shape=[jax.ShapeDtypeStruct((n,), jnp.float32)] * 3
        + [jax.ShapeDtypeStruct((n, d), jnp.float32)],
    )(x, w_att_l, w_att_r, deg_parts)

    parts = _edge_kernel(n, d, w, n_win, e_per_w, n_chunks, chunks_per_tile)(
        row, col, xp, al, ar)

    out = pl.pallas_call(
        _final_body,
        out_shape=jax.ShapeDtypeStruct((n, d), jnp.float32),
    )(parts, x, x_0, al, ar, ds)
    return out


# speedup vs baseline: 41.9157x; 1.1712x over previous
import functools

import jax
import jax.numpy as jnp
from jax import lax
from jax.experimental import pallas as pl
from jax.experimental.pallas import tpu as pltpu
from jax.experimental.pallas import tpu_sc as plsc

_EPS = 0.1
_NC = 2
_NS = 16
_NW = _NC * _NS
_CH = 80
_SC_PARAMS = pltpu.CompilerParams(needs_layout_passes=False)


def _tanh(z):
    a = jnp.abs(z)
    em = jnp.exp(-2.0 * a)
    return jnp.sign(z) * (1.0 - em) / (1.0 + em)


def _deg_kernel(n, w, n_win, e_per_w):
    mesh = plsc.VectorSubcoreMesh(core_axis_name="c", subcore_axis_name="s")

    @functools.partial(
        pl.kernel,
        out_type=jax.ShapeDtypeStruct((_NW * n,), jnp.float32),
        mesh=mesh,
        scratch_types=[
            pltpu.VMEM((e_per_w,), jnp.int32),
            pltpu.VMEM((n,), jnp.float32),
            pltpu.SemaphoreType.DMA,
        ],
        compiler_params=_SC_PARAMS,
    )
    def deg_kernel(col_hbm, out_hbm, idx, hist, smi):
        cid = lax.axis_index("c")
        sid = lax.axis_index("s")
        wid = cid * _NS + sid
        ebase = wid * e_per_w
        ones16 = jnp.ones((16,), jnp.float32)

        pltpu.async_copy(col_hbm.at[pl.ds(ebase, e_per_w)], idx, smi)

        @pl.loop(0, n, step=128)
        def _(i):
            for k in range(8):
                hist[pl.ds(i + k * 16, 16)] = jnp.zeros((16,), jnp.float32)

        pltpu.make_async_copy(col_hbm.at[pl.ds(ebase, e_per_w)], idx, smi).wait()

        @pl.loop(0, e_per_w, step=80)
        def _(g):
            for k in range(5):
                plsc.addupdate_scatter(hist, [idx[pl.ds(g + k * 16, 16)]], ones16)

        pltpu.sync_copy(hist, out_hbm.at[pl.ds(wid * n, n)])

    return deg_kernel


def _scal_body(x_ref, wl_ref, wr_ref, degp_ref, al_ref, ar_ref, ds_ref, xp_ref):
    x = x_ref[...]
    al_ref[...] = jnp.sum(x * wl_ref[...][None, :], axis=1)
    ar_ref[...] = jnp.sum(x * wr_ref[...][None, :], axis=1)
    deg = jnp.sum(degp_ref[...], axis=0) + 1.0
    dis = lax.rsqrt(deg)
    ds_ref[...] = dis
    xp_ref[...] = x * dis[:, None]


def _edge_kernel(n, d, w, n_win, e_per_w, n_chunks, chunks_per_tile):
    mesh = plsc.VectorSubcoreMesh(core_axis_name="c", subcore_axis_name="s")

    @functools.partial(
        pl.kernel,
        out_type=jax.ShapeDtypeStruct((_NC, n, d), jnp.float32),
        mesh=mesh,
        scratch_types=[
            pltpu.VMEM((n,), jnp.float32),
            pltpu.VMEM((n,), jnp.float32),
            pltpu.VMEM((w,), jnp.int32),
            pltpu.VMEM((w,), jnp.int32),
            pltpu.VMEM((w,), jnp.int32),
            pltpu.VMEM((w,), jnp.int32),
            pltpu.VMEM((w, d), jnp.float32),
            pltpu.VMEM((w, d), jnp.float32),
            pltpu.VMEM_SHARED((n, d), jnp.float32),
            pltpu.SemaphoreType.DMA,
            pltpu.SemaphoreType.DMA,
            pltpu.SemaphoreType.DMA,
            pltpu.SemaphoreType.DMA,
        ],
        compiler_params=_SC_PARAMS,
    )
    def edge_kernel(row_hbm, col_hbm, xp_hbm, al_hbm, ar_hbm, out_hbm,
                    tab_al, tab_ar, ir0, ir1, ic0, ic1,
                    rw0, rw1, acc, smi0, smi1, smg0, smg1):
        cid = lax.axis_index("c")
        sid = lax.axis_index("s")
        wid = cid * _NS + sid
        ebase = wid * e_per_w

        bufs = ((ir0, ic0, rw0, smi0, smg0),
                (ir1, ic1, rw1, smi1, smg1))

        def start_idx(iw, b):
            ir, ic, _, smi, _ = bufs[b]
            base = ebase + iw * w
            pltpu.async_copy(row_hbm.at[pl.ds(base, w)], ir, smi)
            pltpu.async_copy(col_hbm.at[pl.ds(base, w)], ic, smi)

        def wait_idx(iw, b):
            ir, ic, _, smi, _ = bufs[b]
            base = ebase + iw * w
            pltpu.make_async_copy(row_hbm.at[pl.ds(base, w)], ir, smi).wait()
            pltpu.make_async_copy(col_hbm.at[pl.ds(base, w)], ic, smi).wait()

        def start_gather(b):
            ir, _, rw, _, smg = bufs[b]
            pltpu.async_copy(xp_hbm.at[ir], rw, smg)

        def wait_gather(b):
            ir, _, rw, _, smg = bufs[b]
            pltpu.make_async_copy(xp_hbm.at[ir], rw, smg).wait()

        pltpu.sync_copy(al_hbm, tab_al)
        pltpu.sync_copy(ar_hbm, tab_ar)

        @pl.loop(0, _CH)
        def _(r):
            for k in range(d // 16):
                rw0[r, pl.ds(k * 16, 16)] = jnp.zeros((16,), jnp.float32)

        @pl.loop(0, chunks_per_tile)
        def _(j):
            c = sid * chunks_per_tile + j

            @pl.when(c < n_chunks)
            def _():
                pltpu.sync_copy(rw0, acc.at[pl.ds(c * _CH, _CH)])

        plsc.subcore_barrier()

        start_idx(0, 0)
        start_idx(1, 1)
        wait_idx(0, 0)
        start_gather(0)

        def body(iw, b):
            ir, ic, rw, _, _ = bufs[b]

            wait_gather(b)

            @pl.loop(0, w, step=16)
            def _(g):
                rv = ir[pl.ds(g, 16)]
                cv = ic[pl.ds(g, 16)]
                al = plsc.load_gather(tab_al, [rv])
                ar = plsc.load_gather(tab_ar, [cv])
                t = _tanh(al + ar)
                for j in range(16):
                    cb = t.at[jnp.full((16,), j, jnp.int32)].get(
                        mode="promise_in_bounds")
                    for k in range(d // 16):
                        rw[g + j, pl.ds(k * 16, 16)] = (
                            rw[g + j, pl.ds(k * 16, 16)] * cb)

            @pl.when(iw + 1 < n_win)
            def _():
                wait_idx(iw + 1, 1 - b)
                start_gather(1 - b)

            pltpu.sync_copy(rw, acc.at[ic], add=True)

            @pl.when(iw + 2 < n_win)
            def _():
                start_idx(iw + 2, b)

        @pl.loop(0, (n_win + 1) // 2)
        def _(i):
            for db in range(2):
                iw = 2 * i + db

                @pl.when(iw < n_win)
                def _():
                    body(iw, db)

        plsc.subcore_barrier()

        @pl.loop(0, chunks_per_tile)
        def _(j):
            c = sid * chunks_per_tile + j

            @pl.when(c < n_chunks)
            def _():
                pltpu.sync_copy(acc.at[pl.ds(c * _CH, _CH)], rw0)
                pltpu.sync_copy(rw0, out_hbm.at[cid, pl.ds(c * _CH, _CH)])

    return edge_kernel


def _final_body(p_ref, x_ref, x0_ref, al_ref, ar_ref, ds_ref, o_ref):
    dis = ds_ref[...]
    c = jnp.tanh(al_ref[...] + ar_ref[...]) * dis * dis
    o_ref[...] = ((p_ref[0] + p_ref[1]) * dis[:, None] + x_ref[...] * c[:, None]
                  + _EPS * x0_ref[...])


@jax.jit
def kernel(x, x_0, edge_index, w_att_l, w_att_r):
    n, d = x.shape
    e = edge_index.shape[1]
    e_per_w = e // _NW
    w = 80
    n_win = e_per_w // w
    n_chunks = n // _CH
    chunks_per_tile = -(-n_chunks // _NS)

    row = edge_index[0]
    col = edge_index[1]

    deg_flat = _deg_kernel(n, w, n_win, e_per_w)(col)
    deg_parts = deg_flat.reshape(_NW, n)

    al, ar, ds, xp = pl.pallas_call(
        _scal_body,
        out_shape=[jax.ShapeDtypeStruct((n,), jnp.float32)] * 3
        + [jax.ShapeDtypeStruct((n, d), jnp.float32)],
    )(x, w_att_l, w_att_r, deg_parts)

    parts = _edge_kernel(n, d, w, n_win, e_per_w, n_chunks, chunks_per_tile)(
        row, col, xp, al, ar)

    out = pl.pallas_call(
        _final_body,
        out_shape=jax.ShapeDtypeStruct((n, d), jnp.float32),
    )(parts, x, x_0, al, ar, ds)
    return out


# speedup vs baseline: 41.9715x; 1.0013x over previous
import functools

import jax
import jax.numpy as jnp
from jax import lax
from jax.experimental import pallas as pl
from jax.experimental.pallas import tpu as pltpu
from jax.experimental.pallas import tpu_sc as plsc

_EPS = 0.1
_NC = 2
_NS = 16
_NW = _NC * _NS
_CH = 80
_SC_PARAMS = pltpu.CompilerParams(needs_layout_passes=False)


def _tanh(z):
    a = jnp.abs(z)
    em = jnp.exp(-2.0 * a)
    return jnp.sign(z) * (1.0 - em) / (1.0 + em)


def _deg_kernel(n, w, n_win, e_per_w):
    mesh = plsc.VectorSubcoreMesh(core_axis_name="c", subcore_axis_name="s")

    @functools.partial(
        pl.kernel,
        out_type=jax.ShapeDtypeStruct((_NW * n,), jnp.float32),
        mesh=mesh,
        scratch_types=[
            pltpu.VMEM((e_per_w,), jnp.int32),
            pltpu.VMEM((n,), jnp.float32),
            pltpu.SemaphoreType.DMA,
        ],
        compiler_params=_SC_PARAMS,
    )
    def deg_kernel(col_hbm, out_hbm, idx, hist, smi):
        cid = lax.axis_index("c")
        sid = lax.axis_index("s")
        wid = cid * _NS + sid
        ebase = wid * e_per_w
        ones16 = jnp.ones((16,), jnp.float32)

        pltpu.async_copy(col_hbm.at[pl.ds(ebase, e_per_w)], idx, smi)

        @pl.loop(0, n, step=128)
        def _(i):
            for k in range(8):
                hist[pl.ds(i + k * 16, 16)] = jnp.zeros((16,), jnp.float32)

        pltpu.make_async_copy(col_hbm.at[pl.ds(ebase, e_per_w)], idx, smi).wait()

        @pl.loop(0, e_per_w, step=80)
        def _(g):
            for k in range(5):
                plsc.addupdate_scatter(hist, [idx[pl.ds(g + k * 16, 16)]], ones16)

        pltpu.sync_copy(hist, out_hbm.at[pl.ds(wid * n, n)])

    return deg_kernel


def _scal_body(x_ref, wl_ref, wr_ref, degp_ref, al_ref, ar_ref, ds_ref, xp_ref):
    x = x_ref[...]
    al_ref[...] = jnp.sum(x * wl_ref[...][None, :], axis=1)
    ar_ref[...] = jnp.sum(x * wr_ref[...][None, :], axis=1)
    deg = jnp.sum(degp_ref[...], axis=0) + 1.0
    dis = lax.rsqrt(deg)
    ds_ref[...] = dis
    xp_ref[...] = x * dis[:, None]


def _edge_kernel(n, d, w, n_win, e_per_w, n_chunks, chunks_per_tile):
    mesh = plsc.VectorSubcoreMesh(core_axis_name="c", subcore_axis_name="s")

    @functools.partial(
        pl.kernel,
        out_type=jax.ShapeDtypeStruct((_NC, n, d), jnp.float32),
        mesh=mesh,
        scratch_types=[
            pltpu.VMEM((n,), jnp.float32),
            pltpu.VMEM((n,), jnp.float32),
            pltpu.VMEM((w,), jnp.int32),
            pltpu.VMEM((w,), jnp.int32),
            pltpu.VMEM((w,), jnp.int32),
            pltpu.VMEM((w,), jnp.int32),
            pltpu.VMEM((w, d), jnp.float32),
            pltpu.VMEM((w, d), jnp.float32),
            pltpu.VMEM_SHARED((n, d), jnp.float32),
            pltpu.SemaphoreType.DMA,
            pltpu.SemaphoreType.DMA,
            pltpu.SemaphoreType.DMA,
            pltpu.SemaphoreType.DMA,
        ],
        compiler_params=_SC_PARAMS,
    )
    def edge_kernel(row_hbm, col_hbm, xp_hbm, al_hbm, ar_hbm, out_hbm,
                    tab_al, tab_ar, ir0, ir1, ic0, ic1,
                    rw0, rw1, acc, smi0, smi1, smg0, smg1):
        cid = lax.axis_index("c")
        sid = lax.axis_index("s")
        wid = cid * _NS + sid
        ebase = wid * e_per_w

        bufs = ((ir0, ic0, rw0, smi0, smg0),
                (ir1, ic1, rw1, smi1, smg1))

        def start_idx(iw, b):
            ir, ic, _, smi, _ = bufs[b]
            base = ebase + iw * w
            pltpu.async_copy(row_hbm.at[pl.ds(base, w)], ir, smi)
            pltpu.async_copy(col_hbm.at[pl.ds(base, w)], ic, smi)

        def wait_idx(iw, b):
            ir, ic, _, smi, _ = bufs[b]
            base = ebase + iw * w
            pltpu.make_async_copy(row_hbm.at[pl.ds(base, w)], ir, smi).wait()
            pltpu.make_async_copy(col_hbm.at[pl.ds(base, w)], ic, smi).wait()

        def start_gather(b):
            ir, _, rw, _, smg = bufs[b]
            pltpu.async_copy(xp_hbm.at[ir], rw, smg)

        def wait_gather(b):
            ir, _, rw, _, smg = bufs[b]
            pltpu.make_async_copy(xp_hbm.at[ir], rw, smg).wait()

        pltpu.sync_copy(al_hbm, tab_al)
        pltpu.sync_copy(ar_hbm, tab_ar)

        @pl.loop(0, _CH)
        def _(r):
            for k in range(d // 16):
                rw0[r, pl.ds(k * 16, 16)] = jnp.zeros((16,), jnp.float32)

        @pl.loop(0, chunks_per_tile)
        def _(j):
            c = sid * chunks_per_tile + j

            @pl.when(c < n_chunks)
            def _():
                pltpu.sync_copy(rw0, acc.at[pl.ds(c * _CH, _CH)])

        plsc.subcore_barrier()

        start_idx(0, 0)
        start_idx(1, 1)
        wait_idx(0, 0)
        start_gather(0)

        def body(iw, b):
            ir, ic, rw, _, _ = bufs[b]

            wait_gather(b)

            @plsc.parallel_loop(0, w, step=16)
            def _(g):
                rv = ir[pl.ds(g, 16)]
                cv = ic[pl.ds(g, 16)]
                al = plsc.load_gather(tab_al, [rv])
                ar = plsc.load_gather(tab_ar, [cv])
                t = _tanh(al + ar)
                for j in range(16):
                    cb = t.at[jnp.full((16,), j, jnp.int32)].get(
                        mode="promise_in_bounds")
                    for k in range(d // 16):
                        rw[g + j, pl.ds(k * 16, 16)] = (
                            rw[g + j, pl.ds(k * 16, 16)] * cb)

            @pl.when(iw + 1 < n_win)
            def _():
                wait_idx(iw + 1, 1 - b)
                start_gather(1 - b)

            pltpu.sync_copy(rw, acc.at[ic], add=True)

            @pl.when(iw + 2 < n_win)
            def _():
                start_idx(iw + 2, b)

        @pl.loop(0, (n_win + 1) // 2)
        def _(i):
            for db in range(2):
                iw = 2 * i + db

                @pl.when(iw < n_win)
                def _():
                    body(iw, db)

        plsc.subcore_barrier()

        @pl.loop(0, chunks_per_tile)
        def _(j):
            c = sid * chunks_per_tile + j

            @pl.when(c < n_chunks)
            def _():
                pltpu.sync_copy(acc.at[pl.ds(c * _CH, _CH)], rw0)
                pltpu.sync_copy(rw0, out_hbm.at[cid, pl.ds(c * _CH, _CH)])

    return edge_kernel


def _final_body(p_ref, x_ref, x0_ref, al_ref, ar_ref, ds_ref, o_ref):
    dis = ds_ref[...]
    c = jnp.tanh(al_ref[...] + ar_ref[...]) * dis * dis
    o_ref[...] = ((p_ref[0] + p_ref[1]) * dis[:, None] + x_ref[...] * c[:, None]
                  + _EPS * x0_ref[...])


@jax.jit
def kernel(x, x_0, edge_index, w_att_l, w_att_r):
    n, d = x.shape
    e = edge_index.shape[1]
    e_per_w = e // _NW
    w = 80
    n_win = e_per_w // w
    n_chunks = n // _CH
    chunks_per_tile = -(-n_chunks // _NS)

    row = edge_index[0]
    col = edge_index[1]

    deg_flat = _deg_kernel(n, w, n_win, e_per_w)(col)
    deg_parts = deg_flat.reshape(_NW, n)

    al, ar, ds, xp = pl.pallas_call(
        _scal_body,
        out_shape=[jax.ShapeDtypeStruct((n,), jnp.float32)] * 3
        + [jax.ShapeDtypeStruct((n, d), jnp.float32)],
    )(x, w_att_l, w_att_r, deg_parts)

    parts = _edge_kernel(n, d, w, n_win, e_per_w, n_chunks, chunks_per_tile)(
        row, col, xp, al, ar)

    out = pl.pallas_call(
        _final_body,
        out_shape=jax.ShapeDtypeStruct((n, d), jnp.float32),
    )(parts, x, x_0, al, ar, ds)
    return out
